# baseline (speedup 1.0000x reference)
import functools

import jax
import jax.numpy as jnp
from jax import lax
from jax.experimental import pallas as pl
from jax.experimental.pallas import tpu as pltpu
from jax.experimental.pallas import tpu_sc as plsc

_B, _S, _T, _R = 4, 2048, 3, 256
_ALPHA, _NSUB, _MEM = 16, 2, 16
_BS = _B * _S
_V2 = _R * _ALPHA ** 2
_V3 = _R * _ALPHA ** 3
_NC, _NS = 2, 16
_NW = _NC * _NS
_PPW = _BS // _NW

_CH = 1024


def _mesh():
    return plsc.VectorSubcoreMesh(
        core_axis_name="c", subcore_axis_name="s",
        num_cores=_NC, num_subcores=_NS)


@functools.cache
def _sc_presum_fn():
    nrow = _CH * _MEM // 128

    @functools.partial(
        pl.kernel,
        out_type=(jax.ShapeDtypeStruct((_V2 * _MEM // 128, 128), jnp.float32),
                  jax.ShapeDtypeStruct((_V3 * _MEM // 128, 128), jnp.float32)),
        mesh=_mesh(),
        scratch_types=[
            pltpu.VMEM((2, _NSUB, 2, 8, _CH), jnp.float32),
            pltpu.VMEM((2, nrow, 128), jnp.float32),
            pltpu.SemaphoreType.DMA,
            pltpu.SemaphoreType.DMA,
        ],
        compiler_params=pltpu.CompilerParams(
            use_tc_tiling_on_sc=True, needs_layout_passes=False),
    )
    def scp(w2_hbm, w3_hbm, o2_hbm, o3_hbm, buf_v, out_v, isem, osem):
        wid = lax.axis_index("s") * _NC + lax.axis_index("c")
        colbase = lax.shift_left(
            lax.bitwise_and(lax.iota(jnp.int32, 16), 7), 4)
        rowbase = lax.shift_right_logical(lax.iota(jnp.int32, 16), 3)

        def run_phase(w_hbm, o_hbm, nch, vbase):
            def in_cps(k, par):
                cps = []
                for sub in range(_NSUB):
                    for mt in range(2):
                        cps.append(pltpu.make_async_copy(
                            w_hbm.at[sub, pl.ds(mt * 8, 8),
                                     pl.ds(vbase + k * _CH, _CH)],
                            buf_v.at[par, sub, mt], isem))
                return cps

            def out_cp(k, par):
                row0 = pl.multiple_of((vbase + k * _CH) * _MEM // 128, 128)
                return pltpu.make_async_copy(
                    out_v.at[par], o_hbm.at[pl.ds(row0, nrow)], osem)

            for cp in in_cps(0, 0):
                cp.start()

            @pl.loop(0, nch)
            def _chunk(k):
                par = lax.rem(k, 2)

                @pl.when(k <= nch - 2)
                def _():
                    for cp in in_cps(k + 1, 1 - par):
                        cp.start()

                for cp in in_cps(k, par):
                    cp.wait()

                @pl.when(k >= 2)
                def _():
                    out_cp(k - 2, par).wait()

                @pl.loop(0, _CH // 16)
                def _vblk(t):
                    rowv = rowbase + t * 2
                    for m in range(_MEM):
                        mt, r8 = divmod(m, 8)
                        a = buf_v[par, 0, mt, r8, pl.ds(t * 16, 16)]
                        bb = buf_v[par, 1, mt, r8, pl.ds(t * 16, 16)]
                        plsc.store_scatter(
                            out_v.at[par], [rowv, colbase + m], a + bb)

                out_cp(k, par).start()

            @pl.when(nch >= 2)
            def _():
                out_cp(nch - 2, lax.rem(nch, 2)).wait()
            out_cp(nch - 1, lax.rem(nch + 1, 2)).wait()

        run_phase(w3_hbm, o3_hbm, _V3 // _NW // _CH, wid * (_V3 // _NW))
        run_phase(w2_hbm, o2_hbm, _V2 // _NW // _CH, wid * (_V2 // _NW))

    return scp


_G = 4
_NHALF = 2
_PPWH = _PPW // _NHALF
_NIT = _PPWH // _G


@functools.cache
def _sc_gather_fn(half):
    @functools.partial(
        pl.kernel,
        out_type=jax.ShapeDtypeStruct((_BS // _NHALF, 2 * _R, _MEM),
                                      jnp.float32),
        mesh=_mesh(),
        scratch_types=[
            pltpu.VMEM((2, _G, _T, _R), jnp.int32),
            pltpu.VMEM((2, 4 * _G, 128), jnp.int32),
            pltpu.VMEM((2, _G, 2 * _R, _MEM), jnp.float32),
            pltpu.SemaphoreType.DMA,
            pltpu.SemaphoreType.DMA,
            pltpu.SemaphoreType.DMA,
        ],
        compiler_params=pltpu.CompilerParams(use_tc_tiling_on_sc=False),
    )
    def sc(codes_hbm, w2_hbm, w3_hbm, out_hbm, codes_v, idx_v, rows_v,
           csem, gsem, osem):
        wid = lax.axis_index("s") * _NC + lax.axis_index("c")
        base = half * (_BS // _NHALF) + wid * _PPWH
        obase = wid * _PPWH
        b = lax.shift_right_logical(base, 11)
        s0 = lax.bitwise_and(base, _S - 1)

        def codes_cp(g, par):
            return pltpu.make_async_copy(
                codes_hbm.at[b, pl.ds(s0 + g * _G, _G)], codes_v.at[par], csem)

        def write_cp(g, par):
            return pltpu.make_async_copy(
                rows_v.at[par], out_hbm.at[pl.ds(obase + g * _G, _G)], osem)

        def gathers_drain(g, par):
            return pltpu.make_async_copy(
                out_hbm.at[pl.ds(obase + g * _G, _G)], rows_v.at[par], gsem)

        def compute_idx(par):
            for q in range(_G):
                for i in range(_R // 16):
                    r16 = lax.iota(jnp.int32, 16) + i * 16
                    c0 = codes_v[par, q, 0, pl.ds(i * 16, 16)]
                    c1 = codes_v[par, q, 1, pl.ds(i * 16, 16)]
                    c2 = codes_v[par, q, 2, pl.ds(i * 16, 16)]
                    idx3 = ((r16 * _ALPHA + c2) * _ALPHA + c1) * _ALPHA + c0
                    idx2 = lax.shift_right_logical(idx3 - c0, 4)
                    row, col = divmod(i, 8)
                    idx_v[par, 4 * q + row, pl.ds(col * 16, 16)] = idx2
                    idx_v[par, 4 * q + 2 + row, pl.ds(col * 16, 16)] = idx3

        def fire_gathers(par):
            for q in range(_G):
                pltpu.async_copy(w2_hbm.at[idx_v.at[par, 4 * q]],
                                 rows_v.at[par, q, pl.ds(0, 128)], gsem)
                pltpu.async_copy(w2_hbm.at[idx_v.at[par, 4 * q + 1]],
                                 rows_v.at[par, q, pl.ds(128, 128)], gsem)
                pltpu.async_copy(w3_hbm.at[idx_v.at[par, 4 * q + 2]],
                                 rows_v.at[par, q, pl.ds(256, 128)], gsem)
                pltpu.async_copy(w3_hbm.at[idx_v.at[par, 4 * q + 3]],
                                 rows_v.at[par, q, pl.ds(384, 128)], gsem)

        codes_cp(0, 0).start()
        codes_cp(0, 0).wait()
        compute_idx(0)
        fire_gathers(0)
        codes_cp(1, 1).start()

        @pl.loop(0, _NIT - 1)
        def _g_loop(g):
            par = lax.rem(g, 2)
            parn = 1 - par
            codes_cp(g + 1, parn).wait()
            compute_idx(parn)

            @pl.when(g >= 1)
            def _():
                write_cp(g - 1, parn).wait()

            fire_gathers(parn)

            @pl.when(g <= _NIT - 3)
            def _():
                codes_cp(g + 2, par).start()

            gathers_drain(g, par).wait()
            write_cp(g, par).start()

        parl = (_NIT - 1) % 2
        gathers_drain(_NIT - 1, parl).wait()
        write_cp(_NIT - 1, parl).start()
        write_cp(_NIT - 2, 1 - parl).wait()
        write_cp(_NIT - 1, parl).wait()

    return sc


@jax.jit
def kernel(route_codes_bstr, W_ngram_2, W_ngram_3):
    wt2 = jnp.transpose(W_ngram_2, (0, 2, 1))
    wt3 = jnp.transpose(W_ngram_3, (0, 2, 1))
    o2, o3 = _sc_presum_fn()(wt2, wt3)
    w2s = o2.reshape(_V2, _MEM)
    w3s = o3.reshape(_V3, _MEM)
    halves = [
        _sc_gather_fn(h)(route_codes_bstr, w2s, w3s).reshape(
            _B // _NHALF, _S, 2 * _R * _MEM)
        for h in range(_NHALF)
    ]
    return jnp.concatenate(halves, axis=0)


# speedup vs baseline: 1.6188x; 1.6188x over previous
import functools

import jax
import jax.numpy as jnp
from jax import lax
from jax.experimental import pallas as pl
from jax.experimental.pallas import tpu as pltpu
from jax.experimental.pallas import tpu_sc as plsc

_B, _S, _T, _R = 4, 2048, 3, 256
_ALPHA, _NSUB, _MEM = 16, 2, 16
_BS = _B * _S
_V2 = _R * _ALPHA ** 2
_V3 = _R * _ALPHA ** 3
_NC, _NS = 2, 16
_NW = _NC * _NS
_PPW = _BS // _NW

_CH = 1024


def _mesh():
    return plsc.VectorSubcoreMesh(
        core_axis_name="c", subcore_axis_name="s",
        num_cores=_NC, num_subcores=_NS)


@functools.cache
def _sc_presum_fn():
    nrow = _CH * _MEM // 128

    @functools.partial(
        pl.kernel,
        out_type=(jax.ShapeDtypeStruct((_V2 * _MEM // 128, 128), jnp.float32),
                  jax.ShapeDtypeStruct((_V3 * _MEM // 128, 128), jnp.float32)),
        mesh=_mesh(),
        scratch_types=[
            pltpu.VMEM((2, _NSUB, 2, 8, _CH), jnp.float32),
            pltpu.VMEM((2, nrow, 128), jnp.float32),
            pltpu.SemaphoreType.DMA,
            pltpu.SemaphoreType.DMA,
        ],
        compiler_params=pltpu.CompilerParams(
            use_tc_tiling_on_sc=True, needs_layout_passes=False),
    )
    def scp(w2_hbm, w3_hbm, o2_hbm, o3_hbm, buf_v, out_v, isem, osem):
        wid = lax.axis_index("s") * _NC + lax.axis_index("c")
        colbase = lax.shift_left(
            lax.bitwise_and(lax.iota(jnp.int32, 16), 7), 4)
        rowbase = lax.shift_right_logical(lax.iota(jnp.int32, 16), 3)

        def run_phase(w_hbm, o_hbm, nch, vbase):
            def in_cps(k, par):
                cps = []
                for sub in range(_NSUB):
                    for mt in range(2):
                        cps.append(pltpu.make_async_copy(
                            w_hbm.at[sub, pl.ds(mt * 8, 8),
                                     pl.ds(vbase + k * _CH, _CH)],
                            buf_v.at[par, sub, mt], isem))
                return cps

            def out_cp(k, par):
                row0 = pl.multiple_of((vbase + k * _CH) * _MEM // 128, 128)
                return pltpu.make_async_copy(
                    out_v.at[par], o_hbm.at[pl.ds(row0, nrow)], osem)

            for cp in in_cps(0, 0):
                cp.start()

            @pl.loop(0, nch)
            def _chunk(k):
                par = lax.rem(k, 2)

                @pl.when(k <= nch - 2)
                def _():
                    for cp in in_cps(k + 1, 1 - par):
                        cp.start()

                for cp in in_cps(k, par):
                    cp.wait()

                @pl.when(k >= 2)
                def _():
                    out_cp(k - 2, par).wait()

                @pl.loop(0, _CH // 16)
                def _vblk(t):
                    rowv = rowbase + t * 2
                    for m in range(_MEM):
                        mt, r8 = divmod(m, 8)
                        a = buf_v[par, 0, mt, r8, pl.ds(t * 16, 16)]
                        bb = buf_v[par, 1, mt, r8, pl.ds(t * 16, 16)]
                        plsc.store_scatter(
                            out_v.at[par], [rowv, colbase + m], a + bb)

                out_cp(k, par).start()

            @pl.when(nch >= 2)
            def _():
                out_cp(nch - 2, lax.rem(nch, 2)).wait()
            out_cp(nch - 1, lax.rem(nch + 1, 2)).wait()

        run_phase(w3_hbm, o3_hbm, _V3 // _NW // _CH, wid * (_V3 // _NW))
        run_phase(w2_hbm, o2_hbm, _V2 // _NW // _CH, wid * (_V2 // _NW))

    return scp


_G = 4
_NHALF = 1
_PPWH = _PPW // _NHALF
_NIT = _PPWH // _G


@functools.cache
def _sc_gather_fn(half):
    @functools.partial(
        pl.kernel,
        out_type=jax.ShapeDtypeStruct((_BS // _NHALF, 2 * _R, _MEM),
                                      jnp.float32),
        mesh=_mesh(),
        scratch_types=[
            pltpu.VMEM((2, _G, _T, _R), jnp.int32),
            pltpu.VMEM((2, 4 * _G, 128), jnp.int32),
            pltpu.VMEM((2, _G, 2 * _R, _MEM), jnp.float32),
            pltpu.SemaphoreType.DMA,
            pltpu.SemaphoreType.DMA,
            pltpu.SemaphoreType.DMA,
        ],
        compiler_params=pltpu.CompilerParams(use_tc_tiling_on_sc=False),
    )
    def sc(codes_hbm, w2_hbm, w3_hbm, out_hbm, codes_v, idx_v, rows_v,
           csem, gsem, osem):
        wid = lax.axis_index("s") * _NC + lax.axis_index("c")
        base = half * (_BS // _NHALF) + wid * _PPWH
        obase = wid * _PPWH
        b = lax.shift_right_logical(base, 11)
        s0 = lax.bitwise_and(base, _S - 1)

        def codes_cp(g, par):
            return pltpu.make_async_copy(
                codes_hbm.at[b, pl.ds(s0 + g * _G, _G)], codes_v.at[par], csem)

        def write_cp(g, par):
            return pltpu.make_async_copy(
                rows_v.at[par], out_hbm.at[pl.ds(obase + g * _G, _G)], osem)

        def gathers_drain(g, par):
            return pltpu.make_async_copy(
                out_hbm.at[pl.ds(obase + g * _G, _G)], rows_v.at[par], gsem)

        def compute_idx(par):
            for q in range(_G):
                for i in range(_R // 16):
                    r16 = lax.iota(jnp.int32, 16) + i * 16
                    c0 = codes_v[par, q, 0, pl.ds(i * 16, 16)]
                    c1 = codes_v[par, q, 1, pl.ds(i * 16, 16)]
                    c2 = codes_v[par, q, 2, pl.ds(i * 16, 16)]
                    idx3 = ((r16 * _ALPHA + c2) * _ALPHA + c1) * _ALPHA + c0
                    idx2 = lax.shift_right_logical(idx3 - c0, 4)
                    row, col = divmod(i, 8)
                    idx_v[par, 4 * q + row, pl.ds(col * 16, 16)] = idx2
                    idx_v[par, 4 * q + 2 + row, pl.ds(col * 16, 16)] = idx3

        def fire_gathers(par):
            for q in range(_G):
                pltpu.async_copy(w2_hbm.at[idx_v.at[par, 4 * q]],
                                 rows_v.at[par, q, pl.ds(0, 128)], gsem)
                pltpu.async_copy(w2_hbm.at[idx_v.at[par, 4 * q + 1]],
                                 rows_v.at[par, q, pl.ds(128, 128)], gsem)
                pltpu.async_copy(w3_hbm.at[idx_v.at[par, 4 * q + 2]],
                                 rows_v.at[par, q, pl.ds(256, 128)], gsem)
                pltpu.async_copy(w3_hbm.at[idx_v.at[par, 4 * q + 3]],
                                 rows_v.at[par, q, pl.ds(384, 128)], gsem)

        codes_cp(0, 0).start()
        codes_cp(0, 0).wait()
        compute_idx(0)
        fire_gathers(0)
        codes_cp(1, 1).start()

        @pl.loop(0, _NIT - 1)
        def _g_loop(g):
            par = lax.rem(g, 2)
            parn = 1 - par
            codes_cp(g + 1, parn).wait()
            compute_idx(parn)

            @pl.when(g >= 1)
            def _():
                write_cp(g - 1, parn).wait()

            fire_gathers(parn)

            @pl.when(g <= _NIT - 3)
            def _():
                codes_cp(g + 2, par).start()

            gathers_drain(g, par).wait()
            write_cp(g, par).start()

        parl = (_NIT - 1) % 2
        gathers_drain(_NIT - 1, parl).wait()
        write_cp(_NIT - 1, parl).start()
        write_cp(_NIT - 2, 1 - parl).wait()
        write_cp(_NIT - 1, parl).wait()

    return sc


@jax.jit
def kernel(route_codes_bstr, W_ngram_2, W_ngram_3):
    wt2 = jnp.transpose(W_ngram_2, (0, 2, 1))
    wt3 = jnp.transpose(W_ngram_3, (0, 2, 1))
    o2, o3 = _sc_presum_fn()(wt2, wt3)
    w2s = o2.reshape(_V2, _MEM)
    w3s = o3.reshape(_V3, _MEM)
    halves = [
        _sc_gather_fn(h)(route_codes_bstr, w2s, w3s).reshape(
            _B // _NHALF, _S, 2 * _R * _MEM)
        for h in range(_NHALF)
    ]
    return jnp.concatenate(halves, axis=0)


# speedup vs baseline: 1.6207x; 1.0011x over previous
import functools

import jax
import jax.numpy as jnp
from jax import lax
from jax.experimental import pallas as pl
from jax.experimental.pallas import tpu as pltpu
from jax.experimental.pallas import tpu_sc as plsc

_B, _S, _T, _R = 4, 2048, 3, 256
_ALPHA, _NSUB, _MEM = 16, 2, 16
_BS = _B * _S
_V2 = _R * _ALPHA ** 2
_V3 = _R * _ALPHA ** 3
_NC, _NS = 2, 16
_NW = _NC * _NS
_PPW = _BS // _NW

_CH = 1024


def _mesh():
    return plsc.VectorSubcoreMesh(
        core_axis_name="c", subcore_axis_name="s",
        num_cores=_NC, num_subcores=_NS)


@functools.cache
def _sc_presum_fn():
    nrow = _CH * _MEM // 128

    @functools.partial(
        pl.kernel,
        out_type=(jax.ShapeDtypeStruct((_V2 * _MEM // 128, 128), jnp.float32),
                  jax.ShapeDtypeStruct((_V3 * _MEM // 128, 128), jnp.float32)),
        mesh=_mesh(),
        scratch_types=[
            pltpu.VMEM((2, _NSUB, 2, 8, _CH), jnp.float32),
            pltpu.VMEM((2, nrow, 128), jnp.float32),
            pltpu.SemaphoreType.DMA,
            pltpu.SemaphoreType.DMA,
        ],
        compiler_params=pltpu.CompilerParams(
            use_tc_tiling_on_sc=True, needs_layout_passes=False),
    )
    def scp(w2_hbm, w3_hbm, o2_hbm, o3_hbm, buf_v, out_v, isem, osem):
        wid = lax.axis_index("s") * _NC + lax.axis_index("c")
        colbase = lax.shift_left(
            lax.bitwise_and(lax.iota(jnp.int32, 16), 7), 4)
        rowbase = lax.shift_right_logical(lax.iota(jnp.int32, 16), 3)

        def run_phase(w_hbm, o_hbm, nch, vbase):
            def in_cps(k, par):
                cps = []
                for sub in range(_NSUB):
                    for mt in range(2):
                        cps.append(pltpu.make_async_copy(
                            w_hbm.at[sub, pl.ds(mt * 8, 8),
                                     pl.ds(vbase + k * _CH, _CH)],
                            buf_v.at[par, sub, mt], isem))
                return cps

            def out_cp(k, par):
                row0 = pl.multiple_of((vbase + k * _CH) * _MEM // 128, 128)
                return pltpu.make_async_copy(
                    out_v.at[par], o_hbm.at[pl.ds(row0, nrow)], osem)

            for cp in in_cps(0, 0):
                cp.start()

            @pl.loop(0, nch)
            def _chunk(k):
                par = lax.rem(k, 2)

                @pl.when(k <= nch - 2)
                def _():
                    for cp in in_cps(k + 1, 1 - par):
                        cp.start()

                for cp in in_cps(k, par):
                    cp.wait()

                @pl.when(k >= 2)
                def _():
                    out_cp(k - 2, par).wait()

                @pl.loop(0, _CH // 16)
                def _vblk(t):
                    rowv = rowbase + t * 2
                    for m in range(_MEM):
                        mt, r8 = divmod(m, 8)
                        a = buf_v[par, 0, mt, r8, pl.ds(t * 16, 16)]
                        bb = buf_v[par, 1, mt, r8, pl.ds(t * 16, 16)]
                        plsc.store_scatter(
                            out_v.at[par], [rowv, colbase + m], a + bb)

                out_cp(k, par).start()

            @pl.when(nch >= 2)
            def _():
                out_cp(nch - 2, lax.rem(nch, 2)).wait()
            out_cp(nch - 1, lax.rem(nch + 1, 2)).wait()

        run_phase(w3_hbm, o3_hbm, _V3 // _NW // _CH, wid * (_V3 // _NW))
        run_phase(w2_hbm, o2_hbm, _V2 // _NW // _CH, wid * (_V2 // _NW))

    return scp


_G = 4
_NHALF = 1
_PPWH = _PPW // _NHALF
_NIT = _PPWH // _G


@functools.cache
def _sc_gather_fn(half):
    @functools.partial(
        pl.kernel,
        out_type=jax.ShapeDtypeStruct((_BS // _NHALF, 2 * _R, _MEM),
                                      jnp.float32),
        mesh=_mesh(),
        scratch_types=[
            pltpu.VMEM((2, _G, _T, _R), jnp.int32),
            pltpu.VMEM((2, 4 * _G, 128), jnp.int32),
            pltpu.VMEM((3, _G, 2 * _R, _MEM), jnp.float32),
            pltpu.SemaphoreType.DMA,
            pltpu.SemaphoreType.DMA,
            pltpu.SemaphoreType.DMA,
        ],
        compiler_params=pltpu.CompilerParams(use_tc_tiling_on_sc=False),
    )
    def sc(codes_hbm, w2_hbm, w3_hbm, out_hbm, codes_v, idx_v, rows_v,
           csem, gsem, osem):
        wid = lax.axis_index("s") * _NC + lax.axis_index("c")
        base = half * (_BS // _NHALF) + wid * _PPWH
        obase = wid * _PPWH
        b = lax.shift_right_logical(base, 11)
        s0 = lax.bitwise_and(base, _S - 1)

        def codes_cp(g, par):
            return pltpu.make_async_copy(
                codes_hbm.at[b, pl.ds(s0 + g * _G, _G)], codes_v.at[par], csem)

        def write_cp(g, slot):
            return pltpu.make_async_copy(
                rows_v.at[slot], out_hbm.at[pl.ds(obase + g * _G, _G)], osem)

        def gathers_drain(g, slot):
            return pltpu.make_async_copy(
                out_hbm.at[pl.ds(obase + g * _G, _G)], rows_v.at[slot], gsem)

        def compute_idx(par):
            for q in range(_G):
                for i in range(_R // 16):
                    r16 = lax.iota(jnp.int32, 16) + i * 16
                    c0 = codes_v[par, q, 0, pl.ds(i * 16, 16)]
                    c1 = codes_v[par, q, 1, pl.ds(i * 16, 16)]
                    c2 = codes_v[par, q, 2, pl.ds(i * 16, 16)]
                    idx3 = ((r16 * _ALPHA + c2) * _ALPHA + c1) * _ALPHA + c0
                    idx2 = lax.shift_right_logical(idx3 - c0, 4)
                    row, col = divmod(i, 8)
                    idx_v[par, 4 * q + row, pl.ds(col * 16, 16)] = idx2
                    idx_v[par, 4 * q + 2 + row, pl.ds(col * 16, 16)] = idx3

        def fire_gathers(par, slot):
            for q in range(_G):
                pltpu.async_copy(w2_hbm.at[idx_v.at[par, 4 * q]],
                                 rows_v.at[slot, q, pl.ds(0, 128)], gsem)
                pltpu.async_copy(w2_hbm.at[idx_v.at[par, 4 * q + 1]],
                                 rows_v.at[slot, q, pl.ds(128, 128)], gsem)
                pltpu.async_copy(w3_hbm.at[idx_v.at[par, 4 * q + 2]],
                                 rows_v.at[slot, q, pl.ds(256, 128)], gsem)
                pltpu.async_copy(w3_hbm.at[idx_v.at[par, 4 * q + 3]],
                                 rows_v.at[slot, q, pl.ds(384, 128)], gsem)

        codes_cp(0, 0).start()
        codes_cp(0, 0).wait()
        compute_idx(0)
        fire_gathers(0, 0)
        codes_cp(1, 1).start()

        @pl.loop(0, _NIT - 1)
        def _g_loop(g):
            par = lax.rem(g, 2)
            parn = 1 - par
            slot = lax.rem(g, 3)
            slotn = lax.rem(g + 1, 3)
            codes_cp(g + 1, parn).wait()
            compute_idx(parn)

            @pl.when(g >= 2)
            def _():
                write_cp(g - 2, slotn).wait()

            fire_gathers(parn, slotn)

            @pl.when(g <= _NIT - 3)
            def _():
                codes_cp(g + 2, par).start()

            gathers_drain(g, slot).wait()
            write_cp(g, slot).start()

        parl = (_NIT - 1) % 2
        slotl = (_NIT - 1) % 3
        gathers_drain(_NIT - 1, slotl).wait()
        write_cp(_NIT - 1, slotl).start()
        write_cp(_NIT - 3, (_NIT - 3) % 3).wait()
        write_cp(_NIT - 2, (_NIT - 2) % 3).wait()
        write_cp(_NIT - 1, slotl).wait()

    return sc


@jax.jit
def kernel(route_codes_bstr, W_ngram_2, W_ngram_3):
    wt2 = jnp.transpose(W_ngram_2, (0, 2, 1))
    wt3 = jnp.transpose(W_ngram_3, (0, 2, 1))
    o2, o3 = _sc_presum_fn()(wt2, wt3)
    w2s = o2.reshape(_V2, _MEM)
    w3s = o3.reshape(_V3, _MEM)
    halves = [
        _sc_gather_fn(h)(route_codes_bstr, w2s, w3s).reshape(
            _B // _NHALF, _S, 2 * _R * _MEM)
        for h in range(_NHALF)
    ]
    return jnp.concatenate(halves, axis=0)
